# baseline (speedup 1.0000x reference)
import functools

import jax
import jax.numpy as jnp
from jax import lax
from jax.experimental import pallas as pl
from jax.experimental.pallas import tpu as pltpu
from jax.experimental.pallas import tpu_sc as plsc

_LP = (0.99999705, -0.49982547, 0.33078786, -0.23417359, 0.14810663,
       -0.06577001, 0.01402682)


def _fold_body(We, be, Wf1, bf1, Ws1, bs1, Wf2, bf2, Ws2, bs2, wc, bc):
    we = We[...]
    bev = be[...]
    pieces_w = []
    pieces_b = []
    for Wm, bm in ((Wf1, bf1), (Ws1, bs1), (Wf2, bf2), (Ws2, bs2)):
        wblk = Wm[256:384, :]
        pieces_w.append(jnp.dot(we, wblk, preferred_element_type=jnp.float32))
        pieces_b.append(jnp.dot(bev, wblk, preferred_element_type=jnp.float32)
                        + bm[...])
    wc[...] = jnp.concatenate(pieces_w, axis=1)
    bc[...] = jnp.concatenate(pieces_b, axis=1)


def _edge_body(ea, wc, bc, e1, e2):
    y = jnp.dot(ea[...], wc[...], preferred_element_type=jnp.float32) + bc[...]
    e1[...] = y[:, 0:256]
    e2[...] = y[:, 256:512]


def _node1_body(x, Wn, bn, Wd, Ws_, h0, tabd, tabs):
    h = jnp.dot(x[...], Wn[...], preferred_element_type=jnp.float32) + bn[...]
    h0[...] = h
    tabd[...] = jnp.dot(h, Wd[...], preferred_element_type=jnp.float32)
    tabs[...] = jnp.dot(h, Ws_[...], preferred_element_type=jnp.float32)


def _node2_body(a0, a1, hprev, Wd, Ws_, h1, tabd, tabs):
    h = jnp.maximum(a0[...] + a1[...] + hprev[...], 0.0)
    h1[...] = h
    tabd[...] = jnp.dot(h, Wd[...], preferred_element_type=jnp.float32)
    tabs[...] = jnp.dot(h, Ws_[...], preferred_element_type=jnp.float32)


def _out_body(a0, a1, hprev, Wa, ba, out):
    h = jnp.maximum(a0[...] + a1[...] + hprev[...], 0.0)
    out[...] = jnp.dot(h, Wa[...], preferred_element_type=jnp.float32) + ba[...]


def _full(shape):
    return pl.BlockSpec(shape, lambda i: (0,) * len(shape))


def _rows(bn, cols):
    return pl.BlockSpec((bn, cols), lambda i: (i, 0))


def _sc_edge_layer(tabd, tabs, edge, src, dst, n_nodes):
    n_edges = edge.shape[0]
    H = 128
    C = 40
    n_tiles = 32
    per_tile = n_edges // n_tiles
    n_chunks = per_tile // C
    RPT = (n_nodes // 16) // 8 * 8
    REM = n_nodes - 16 * RPT
    ZR = 16

    mesh = plsc.VectorSubcoreMesh(core_axis_name="c", subcore_axis_name="s")

    @functools.partial(
        pl.kernel, mesh=mesh,
        out_type=(jax.ShapeDtypeStruct((n_nodes, H), jnp.float32),
                  jax.ShapeDtypeStruct((n_nodes, H), jnp.float32)),
        scratch_types=[
            pltpu.VMEM((C,), jnp.int32),
            pltpu.VMEM((C,), jnp.int32),
            pltpu.VMEM((C, 256), jnp.float32),
            pltpu.VMEM((C, 256), jnp.float32),
            pltpu.VMEM((C, 256), jnp.float32),
            pltpu.VMEM((C, H), jnp.float32),
            pltpu.VMEM((ZR, H), jnp.float32),
            pltpu.VMEM_SHARED((n_nodes, H), jnp.float32),
            pltpu.SemaphoreType.DMA,
            pltpu.SemaphoreType.DMA,
        ],
    )
    def k(tabd_h, tabs_h, edge_h, src_h, dst_h, out0, out1,
          dstv, srcv, gd, gs, ge, mv, zv, agg, sem1, sem2):
        c = lax.axis_index("c")
        s = lax.axis_index("s")
        tid = c * 16 + s

        def zrow(r, _):
            for kk in range(H // 16):
                zv[r, pl.ds(kk * 16, 16)] = jnp.zeros((16,), jnp.float32)
            return 0
        lax.fori_loop(0, ZR, zrow, 0)
        row0 = s * RPT
        for zb in range(RPT // ZR):
            pltpu.sync_copy(zv, agg.at[pl.ds(row0 + zb * ZR, ZR)])

        @pl.when(s == 15)
        def _():
            pltpu.sync_copy(zv.at[pl.ds(0, REM)],
                            agg.at[pl.ds(16 * RPT, REM)])
        plsc.subcore_barrier()

        base0 = tid * per_tile

        def chunk(g, _):
            b = base0 + g * C
            pltpu.sync_copy(dst_h.at[pl.ds(b, C)], dstv)
            pltpu.sync_copy(src_h.at[pl.ds(b, C)], srcv)
            cp1 = pltpu.async_copy(tabd_h.at[dstv], gd, sem1)
            cp2 = pltpu.async_copy(tabs_h.at[srcv], gs, sem2)
            pltpu.sync_copy(edge_h.at[pl.ds(b, C)], ge)
            cp1.wait()
            cp2.wait()

            def row(r, _):
                for kk in range(H // 16):
                    slf = pl.ds(kk * 16, 16)
                    sls = pl.ds(H + kk * 16, 16)
                    f = gd[r, slf] + gs[r, slf] + ge[r, slf]
                    sv = gd[r, sls] + gs[r, sls] + ge[r, sls]
                    y = jnp.exp(-jnp.abs(sv))
                    p = _LP[6]
                    for cf in (_LP[5], _LP[4], _LP[3], _LP[2], _LP[1], _LP[0]):
                        p = p * y + cf
                    sp = jnp.maximum(sv, 0.0) + p * y
                    mv[r, slf] = sp / (1.0 + jnp.exp(-f))
                return 0
            lax.fori_loop(0, C, row, 0)
            pltpu.sync_copy(mv, agg.at[dstv], add=True)
            return 0
        lax.fori_loop(0, n_chunks, chunk, 0)
        plsc.subcore_barrier()

        sl = pl.ds(row0, RPT)
        sl_rem = pl.ds(16 * RPT, REM)

        @pl.when(c == 0)
        def _():
            pltpu.sync_copy(agg.at[sl], out0.at[sl])

            @pl.when(s == 15)
            def _():
                pltpu.sync_copy(agg.at[sl_rem], out0.at[sl_rem])

        @pl.when(c == 1)
        def _():
            pltpu.sync_copy(agg.at[sl], out1.at[sl])

            @pl.when(s == 15)
            def _():
                pltpu.sync_copy(agg.at[sl_rem], out1.at[sl_rem])

    return k(tabd, tabs, edge, src, dst)


def kernel(x, edge_index, edge_attr, Wn, bn, We, be,
           Wf1, bf1, Ws1, bs1, Wf2, bf2, Ws2, bs2, Wa, ba):
    n_nodes, dn = x.shape
    n_edges = edge_attr.shape[0]
    H = Wn.shape[1]
    src = edge_index[0]
    dst = edge_index[1]

    bn2 = bn.reshape(1, H)
    be2 = be.reshape(1, We.shape[1])
    ba2 = ba.reshape(1, H)
    bf1r, bs1r = bf1.reshape(1, H), bs1.reshape(1, H)
    bf2r, bs2r = bf2.reshape(1, H), bs2.reshape(1, H)
    Wd1 = jnp.concatenate([Wf1[0:H], Ws1[0:H]], axis=1)
    Wsj1 = jnp.concatenate([Wf1[H:2 * H], Ws1[H:2 * H]], axis=1)
    Wd2 = jnp.concatenate([Wf2[0:H], Ws2[0:H]], axis=1)
    Wsj2 = jnp.concatenate([Wf2[H:2 * H], Ws2[H:2 * H]], axis=1)

    f32 = jnp.float32

    wc, bc = pl.pallas_call(
        _fold_body,
        out_shape=(jax.ShapeDtypeStruct((16, 512), f32),
                   jax.ShapeDtypeStruct((1, 512), f32)),
    )(We, be2, Wf1, bf1r, Ws1, bs1r, Wf2, bf2r, Ws2, bs2r)

    BE = 2000
    edge1, edge2 = pl.pallas_call(
        _edge_body,
        grid=(n_edges // BE,),
        in_specs=[_rows(BE, 16), _full((16, 512)), _full((1, 512))],
        out_specs=(_rows(BE, 256), _rows(BE, 256)),
        out_shape=(jax.ShapeDtypeStruct((n_edges, 256), f32),
                   jax.ShapeDtypeStruct((n_edges, 256), f32)),
    )(edge_attr, wc, bc)

    BN = 2000
    h0, tabd1, tabs1 = pl.pallas_call(
        _node1_body,
        grid=(n_nodes // BN,),
        in_specs=[_rows(BN, dn), _full((dn, H)), _full((1, H)),
                  _full((H, 256)), _full((H, 256))],
        out_specs=(_rows(BN, H), _rows(BN, 256), _rows(BN, 256)),
        out_shape=(jax.ShapeDtypeStruct((n_nodes, H), f32),
                   jax.ShapeDtypeStruct((n_nodes, 256), f32),
                   jax.ShapeDtypeStruct((n_nodes, 256), f32)),
    )(x, Wn, bn2, Wd1, Wsj1)

    a0, a1 = _sc_edge_layer(tabd1, tabs1, edge1, src, dst, n_nodes)

    h1, tabd2, tabs2 = pl.pallas_call(
        _node2_body,
        grid=(n_nodes // BN,),
        in_specs=[_rows(BN, H), _rows(BN, H), _rows(BN, H),
                  _full((H, 256)), _full((H, 256))],
        out_specs=(_rows(BN, H), _rows(BN, 256), _rows(BN, 256)),
        out_shape=(jax.ShapeDtypeStruct((n_nodes, H), f32),
                   jax.ShapeDtypeStruct((n_nodes, 256), f32),
                   jax.ShapeDtypeStruct((n_nodes, 256), f32)),
    )(a0, a1, h0, Wd2, Wsj2)

    b0, b1 = _sc_edge_layer(tabd2, tabs2, edge2, src, dst, n_nodes)

    out = pl.pallas_call(
        _out_body,
        grid=(n_nodes // BN,),
        in_specs=[_rows(BN, H), _rows(BN, H), _rows(BN, H),
                  _full((H, H)), _full((1, H))],
        out_specs=_rows(BN, H),
        out_shape=jax.ShapeDtypeStruct((n_nodes, H), f32),
    )(b0, b1, h1, Wa, ba2)

    return out
